# baseline (speedup 1.0000x reference)
import dataclasses

import jax
import jax.numpy as jnp
from jax import lax
from jax.experimental import pallas as pl
from jax.experimental.pallas import tpu as pltpu
from jax.experimental.pallas import tpu_sc as plsc

_N = 4096
_MAXDEG = 64
_L = 16
_NC, _NS = 2, 16
_NW = _NC * _NS
_RPW = _N // _NW
_RBLK = 4
_NBLK = _RPW // _RBLK
_UNROLL = 8


def _sc_kernel(rank_hbm, c_hbm, mask_hbm, out_hbm,
               c_v, rank_v, g_v, in0, in1, out0, out1,
               sem_in0, sem_in1, sem_out0, sem_out1):
    wid = lax.axis_index("s") * _NC + lax.axis_index("c")
    base = wid * _RPW

    pltpu.sync_copy(c_hbm, c_v)
    pltpu.sync_copy(rank_hbm.at[pl.ds(base, _RPW)], rank_v)

    @pl.loop(0, _RPW, step=_L)
    def _(i):
        rc = jnp.minimum(rank_v[pl.ds(i, _L)], _MAXDEG - 1)
        g_v[pl.ds(i, _L)] = plsc.load_gather(c_v, [rc])

    sem_in = (sem_in0, sem_in1)
    sem_out = (sem_out0, sem_out1)
    in_bufs = (in0, in1)
    out_bufs = (out0, out1)

    def rows(blk):
        return pl.ds(base + blk * _RBLK, _RBLK)

    pltpu.async_copy(mask_hbm.at[rows(0)], in_bufs[0], sem_in[0])
    pltpu.async_copy(mask_hbm.at[rows(1)], in_bufs[1], sem_in[1])

    @pl.loop(0, _NBLK, step=2)
    def _(blk0):
        for b in (0, 1):
            blk = blk0 + b
            pltpu.make_async_copy(mask_hbm.at[rows(blk)], in_bufs[b],
                                  sem_in[b]).wait()

            @pl.when(blk >= 2)
            def _():
                pltpu.make_async_copy(out_bufs[b], out_hbm.at[rows(blk)],
                                      sem_out[b]).wait()

            for row in range(_RBLK):
                ridx = blk * _RBLK + row
                gvec = plsc.load_gather(
                    g_v, [jnp.full((_L,), ridx, jnp.int32)])
                src = in_bufs[b]
                dst = out_bufs[b]

                @pl.loop(0, _N, step=_L * _UNROLL)
                def _(c0):
                    for u in range(_UNROLL):
                        sl = pl.ds(c0 + u * _L, _L)
                        dst[row, sl] = gvec * src[row, sl]

            pltpu.async_copy(out_bufs[b], out_hbm.at[rows(blk)], sem_out[b])

            @pl.when(blk + 2 < _NBLK)
            def _():
                pltpu.async_copy(mask_hbm.at[rows(blk + 2)], in_bufs[b],
                                 sem_in[b])

    for b in (0, 1):
        blk = _NBLK - 2 + b
        pltpu.make_async_copy(out_bufs[b], out_hbm.at[rows(blk)],
                              sem_out[b]).wait()


def kernel(x, rank, sparse_mask, c):
    del x
    mesh = plsc.VectorSubcoreMesh(core_axis_name="c", subcore_axis_name="s")
    cp = pltpu.CompilerParams()
    if "needs_layout_passes" in pltpu.CompilerParams.__dataclass_fields__:
        cp = dataclasses.replace(cp, needs_layout_passes=False)
    kern = pl.kernel(
        _sc_kernel,
        out_type=jax.ShapeDtypeStruct((_N, _N), jnp.float32),
        mesh=mesh,
        scratch_types=[
            pltpu.VMEM((_MAXDEG,), jnp.float32),
            pltpu.VMEM((_RPW,), jnp.int32),
            pltpu.VMEM((_RPW,), jnp.float32),
            pltpu.VMEM((_RBLK, _N), jnp.float32),
            pltpu.VMEM((_RBLK, _N), jnp.float32),
            pltpu.VMEM((_RBLK, _N), jnp.float32),
            pltpu.VMEM((_RBLK, _N), jnp.float32),
            pltpu.SemaphoreType.DMA,
            pltpu.SemaphoreType.DMA,
            pltpu.SemaphoreType.DMA,
            pltpu.SemaphoreType.DMA,
        ],
        compiler_params=cp,
    )
    return kern(rank, c, sparse_mask)
